# baseline (speedup 1.0000x reference)
import jax
import jax.numpy as jnp
from jax.experimental import pallas as pl
from jax.experimental.pallas import tpu as pltpu

_CHUNKS = 6
_ROWS_PER_CHUNK = 8


def _copy_body(x_hbm, o_hbm, buf, in_sems, out_sems):
    for i in range(_CHUNKS):
        r = i * _ROWS_PER_CHUNK
        pltpu.make_async_copy(
            x_hbm.at[pl.ds(r, _ROWS_PER_CHUNK), :],
            buf.at[pl.ds(r, _ROWS_PER_CHUNK), :],
            in_sems.at[i],
        ).start()
    for i in range(_CHUNKS):
        r = i * _ROWS_PER_CHUNK
        pltpu.make_async_copy(
            x_hbm.at[pl.ds(r, _ROWS_PER_CHUNK), :],
            buf.at[pl.ds(r, _ROWS_PER_CHUNK), :],
            in_sems.at[i],
        ).wait()
        pltpu.make_async_copy(
            buf.at[pl.ds(r, _ROWS_PER_CHUNK), :],
            o_hbm.at[pl.ds(r, _ROWS_PER_CHUNK), :],
            out_sems.at[i],
        ).start()
    for i in range(_CHUNKS):
        r = i * _ROWS_PER_CHUNK
        pltpu.make_async_copy(
            buf.at[pl.ds(r, _ROWS_PER_CHUNK), :],
            o_hbm.at[pl.ds(r, _ROWS_PER_CHUNK), :],
            out_sems.at[i],
        ).wait()


def kernel(xyz, features):
    del features
    B, N, C = xyz.shape
    flat = jnp.transpose(xyz, (2, 0, 1)).reshape(C * B, N)
    out = pl.pallas_call(
        _copy_body,
        in_specs=[pl.BlockSpec(memory_space=pltpu.MemorySpace.HBM)],
        out_specs=pl.BlockSpec(memory_space=pltpu.MemorySpace.HBM),
        scratch_shapes=[
            pltpu.VMEM((C * B, N), jnp.float32),
            pltpu.SemaphoreType.DMA((_CHUNKS,)),
            pltpu.SemaphoreType.DMA((_CHUNKS,)),
        ],
        out_shape=jax.ShapeDtypeStruct(flat.shape, flat.dtype),
    )(flat)
    return jnp.transpose(out.reshape(C, B, N), (1, 2, 0))
